# Initial kernel scaffold; baseline (speedup 1.0000x reference)
import jax
import jax.numpy as jnp
from jax.experimental import pallas as pl


def kernel(text, emb, W_ih, W_hh, b_ih, b_hh, W_lin, b_lin):
    raise NotImplementedError("write your pallas kernel here")


# speedup vs baseline: 3.7020x; 3.7020x over previous
import functools

import jax
import jax.numpy as jnp
from jax import lax
from jax.experimental import pallas as pl
from jax.experimental.pallas import tpu as pltpu
from jax.experimental.pallas import tpu_sc as plsc

_CHUNK = 128


def _sc_gather(n_rows, emb_dim, dtype, n_workers, chunks_per_w):
    mesh = plsc.VectorSubcoreMesh(core_axis_name="c", subcore_axis_name="s")
    nc = mesh.num_cores

    @functools.partial(
        pl.kernel,
        out_type=jax.ShapeDtypeStruct((n_rows, emb_dim), dtype),
        mesh=mesh,
        compiler_params=pltpu.CompilerParams(use_tc_tiling_on_sc=False),
        scratch_types=[
            pltpu.VMEM((chunks_per_w, _CHUNK), jnp.int32),
            pltpu.VMEM((2, _CHUNK, emb_dim), dtype),
            pltpu.SemaphoreType.DMA,
            pltpu.SemaphoreType.DMA,
        ],
    )
    def gather_k(table_hbm, idx_hbm, out_hbm, idx_v, rows_v, gsem, ssem):
        wid = lax.axis_index("s") * nc + lax.axis_index("c")
        row_base = wid * chunks_per_w * _CHUNK
        pltpu.sync_copy(idx_hbm.at[pl.ds(wid * chunks_per_w, chunks_per_w)],
                        idx_v)

        def body(i, _):
            pltpu.async_copy(table_hbm.at[idx_v.at[i]], rows_v.at[0],
                             gsem).wait()
            pltpu.sync_copy(rows_v.at[0],
                            out_hbm.at[pl.ds(row_base + i * _CHUNK, _CHUNK)])
            return 0

        lax.fori_loop(0, chunks_per_w, body, 0)

    return gather_k


def _tc_rnn(T, B, E, H, C):
    def body(x_ref, wih_ref, whh_ref, bih_ref, bhh_ref, wlin_ref, blin_ref,
             out_ref, h_ref):
        t = pl.program_id(0)

        @pl.when(t == 0)
        def _():
            h_ref[...] = jnp.zeros_like(h_ref)

        x = x_ref[0]
        h = h_ref[...]
        hn = jnp.tanh(
            jnp.dot(x, wih_ref[...], preferred_element_type=jnp.float32)
            + jnp.dot(h, whh_ref[...], preferred_element_type=jnp.float32)
            + bih_ref[...] + bhh_ref[...])
        h_ref[...] = hn

        @pl.when(t == T - 1)
        def _():
            logits = jnp.dot(hn, wlin_ref[...],
                             preferred_element_type=jnp.float32) + blin_ref[...]
            m = jnp.max(logits, axis=1, keepdims=True)
            e = jnp.exp(logits - m)
            out_ref[...] = e / jnp.sum(e, axis=1, keepdims=True)

    return pl.pallas_call(
        body,
        grid=(T,),
        in_specs=[
            pl.BlockSpec((1, B, E), lambda t: (t, 0, 0)),
            pl.BlockSpec((E, H), lambda t: (0, 0)),
            pl.BlockSpec((H, H), lambda t: (0, 0)),
            pl.BlockSpec((1, H), lambda t: (0, 0)),
            pl.BlockSpec((1, H), lambda t: (0, 0)),
            pl.BlockSpec((H, C), lambda t: (0, 0)),
            pl.BlockSpec((1, C), lambda t: (0, 0)),
        ],
        out_specs=pl.BlockSpec((B, C), lambda t: (0, 0)),
        out_shape=jax.ShapeDtypeStruct((B, C), jnp.float32),
        scratch_shapes=[pltpu.VMEM((B, H), jnp.float32)],
    )


def kernel(text, emb, W_ih, W_hh, b_ih, b_hh, W_lin, b_lin):
    B, T = text.shape
    V, E = emb.shape
    H = W_hh.shape[0]
    C = W_lin.shape[0]

    n_workers = 32
    n = T * B
    chunks_per_w = n // (n_workers * _CHUNK)
    idx = text.T.reshape(n_workers * chunks_per_w, _CHUNK).astype(jnp.int32)

    x_flat = _sc_gather(n, E, emb.dtype, n_workers, chunks_per_w)(emb, idx)
    x_all = x_flat.reshape(T, B, E)

    out = _tc_rnn(T, B, E, H, C)(
        x_all,
        W_ih.T, W_hh.T,
        b_ih.reshape(1, H), b_hh.reshape(1, H),
        W_lin.T, b_lin.reshape(1, C),
    )
    return out


# speedup vs baseline: 4.3608x; 1.1780x over previous
import functools

import jax
import jax.numpy as jnp
from jax import lax
from jax.experimental import pallas as pl
from jax.experimental.pallas import tpu as pltpu
from jax.experimental.pallas import tpu_sc as plsc

_CHUNK = 128


def _sc_gather(n_rows, emb_dim, dtype, n_workers, chunks_per_w):
    mesh = plsc.VectorSubcoreMesh(core_axis_name="c", subcore_axis_name="s")
    nc = mesh.num_cores

    K = 4
    G = chunks_per_w // K
    grp = K * _CHUNK

    @functools.partial(
        pl.kernel,
        out_type=jax.ShapeDtypeStruct((n_rows, emb_dim), dtype),
        mesh=mesh,
        compiler_params=pltpu.CompilerParams(use_tc_tiling_on_sc=False),
        scratch_types=[
            pltpu.VMEM((chunks_per_w, _CHUNK), jnp.int32),
            pltpu.VMEM((2, grp, emb_dim), dtype),
            pltpu.SemaphoreType.DMA,
            pltpu.SemaphoreType.DMA,
            pltpu.SemaphoreType.DMA,
            pltpu.SemaphoreType.DMA,
        ],
    )
    def gather_k(table_hbm, idx_hbm, out_hbm, idx_v, rows_v,
                 gsem0, gsem1, ssem0, ssem1):
        wid = lax.axis_index("s") * nc + lax.axis_index("c")
        row_base = wid * chunks_per_w * _CHUNK
        gsems = (gsem0, gsem1)
        ssems = (ssem0, ssem1)
        pltpu.sync_copy(idx_hbm.at[pl.ds(wid * chunks_per_w, chunks_per_w)],
                        idx_v)

        def fire(g, b):
            for j in range(K):
                pltpu.async_copy(
                    table_hbm.at[idx_v.at[g * K + j]],
                    rows_v.at[b, pl.ds(j * _CHUNK, _CHUNK)],
                    gsems[b])

        def drain(g, b):
            pltpu.make_async_copy(
                out_hbm.at[pl.ds(row_base + g * grp, grp)],
                rows_v.at[b], gsems[b]).wait()

        fire(0, 0)

        def body(jj, _):
            for u in range(2):
                g = jj + u
                b = u
                drain(g, b)

                @pl.when(g + 1 < G)
                def _():
                    @pl.when(g >= 1)
                    def _():
                        pltpu.make_async_copy(
                            rows_v.at[1 - b],
                            out_hbm.at[pl.ds(row_base + (g - 1) * grp, grp)],
                            ssems[1 - b]).wait()

                    fire(g + 1, 1 - b)

                pltpu.async_copy(
                    rows_v.at[b],
                    out_hbm.at[pl.ds(row_base + g * grp, grp)],
                    ssems[b])
            return 0

        lax.fori_loop(0, G // 2, lambda jj, c: body(jj * 2, c), 0)
        pltpu.make_async_copy(
            rows_v.at[0],
            out_hbm.at[pl.ds(row_base + (G - 2) * grp, grp)], ssem0).wait()
        pltpu.make_async_copy(
            rows_v.at[1],
            out_hbm.at[pl.ds(row_base + (G - 1) * grp, grp)], ssem1).wait()

    return gather_k


def _tc_rnn(T, B, E, H, C):
    def body(x_ref, wih_ref, whh_ref, bih_ref, bhh_ref, wlin_ref, blin_ref,
             out_ref, h_ref):
        t = pl.program_id(0)

        @pl.when(t == 0)
        def _():
            h_ref[...] = jnp.zeros_like(h_ref)

        x = x_ref[0]
        h = h_ref[...]
        hn = jnp.tanh(
            jnp.dot(x, wih_ref[...], preferred_element_type=jnp.float32)
            + jnp.dot(h, whh_ref[...], preferred_element_type=jnp.float32)
            + bih_ref[...] + bhh_ref[...])
        h_ref[...] = hn

        @pl.when(t == T - 1)
        def _():
            logits = jnp.dot(hn, wlin_ref[...],
                             preferred_element_type=jnp.float32) + blin_ref[...]
            m = jnp.max(logits, axis=1, keepdims=True)
            e = jnp.exp(logits - m)
            out_ref[...] = e / jnp.sum(e, axis=1, keepdims=True)

    return pl.pallas_call(
        body,
        grid=(T,),
        in_specs=[
            pl.BlockSpec((1, B, E), lambda t: (t, 0, 0)),
            pl.BlockSpec((E, H), lambda t: (0, 0)),
            pl.BlockSpec((H, H), lambda t: (0, 0)),
            pl.BlockSpec((1, H), lambda t: (0, 0)),
            pl.BlockSpec((1, H), lambda t: (0, 0)),
            pl.BlockSpec((H, C), lambda t: (0, 0)),
            pl.BlockSpec((1, C), lambda t: (0, 0)),
        ],
        out_specs=pl.BlockSpec((B, C), lambda t: (0, 0)),
        out_shape=jax.ShapeDtypeStruct((B, C), jnp.float32),
        scratch_shapes=[pltpu.VMEM((B, H), jnp.float32)],
    )


def kernel(text, emb, W_ih, W_hh, b_ih, b_hh, W_lin, b_lin):
    B, T = text.shape
    V, E = emb.shape
    H = W_hh.shape[0]
    C = W_lin.shape[0]

    n_workers = 32
    n = T * B
    chunks_per_w = n // (n_workers * _CHUNK)
    idx = text.T.reshape(n_workers * chunks_per_w, _CHUNK).astype(jnp.int32)

    x_flat = _sc_gather(n, E, emb.dtype, n_workers, chunks_per_w)(emb, idx)
    x_all = x_flat.reshape(T, B, E)

    out = _tc_rnn(T, B, E, H, C)(
        x_all,
        W_ih.T, W_hh.T,
        b_ih.reshape(1, H), b_hh.reshape(1, H),
        W_lin.T, b_lin.reshape(1, C),
    )
    return out
